# Initial kernel scaffold; baseline (speedup 1.0000x reference)
import jax
import jax.numpy as jnp
from jax.experimental import pallas as pl


def kernel(x, edge_index, W1, b1, W2, b2):
    raise NotImplementedError("write your pallas kernel here")


# speedup vs baseline: 228.5761x; 228.5761x over previous
import functools

import jax
import jax.numpy as jnp
import numpy as np
from jax import lax
from jax.experimental import pallas as pl
from jax.experimental.pallas import tpu as pltpu
from jax.experimental.pallas import tpu_sc as plsc

N = 100000
E = 3200000
D = 128
H = 64
C = 16
K = 10
ALPHA = 0.1

NP = 100096
HALF = NP // 2
DUMP = HALF
HROWS = HALF // 16
ZB = 184

NS = 16
TT = 1568
EP = NS * TT * 128
SB = 56
NCH = TT // SB

_mesh = plsc.VectorSubcoreMesh(core_axis_name="c", subcore_axis_name="s")
_sc_params = pltpu.CompilerParams(use_tc_tiling_on_sc=False)
_Z = np.int32(0)


def _i32(v):
    return jnp.asarray(v, jnp.int32)


def _loop32(lo, hi):
    return pl.loop(jnp.int32(lo), jnp.int32(hi))


def _zero_acc(acc_sh, zbuf, sid):
    @_loop32(0, ZB)
    def _(i):
        zbuf[pl.ds(i, 1), :] = jnp.zeros((1, C), jnp.float32)

    @_loop32(0, 17)
    def _(j):
        pltpu.sync_copy(zbuf,
                        acc_sh.at[pl.ds(sid * _i32(HROWS) + j * _i32(ZB),
                                        ZB)])


def _load_dst(dst0_hbm, dst1_hbm, dstv, base, cid):
    @pl.when(cid == 0)
    def _():
        pltpu.sync_copy(dst0_hbm.at[pl.ds(base, SB)], dstv)

    @pl.when(cid == 1)
    def _():
        pltpu.sync_copy(dst1_hbm.at[pl.ds(base, SB)], dstv)


def _writeback(acc_sh, acc_hbm, cid, sid):
    src = acc_sh.at[pl.ds(sid * _i32(HROWS), HROWS)]

    @pl.when(cid == 0)
    def _():
        pltpu.sync_copy(src, acc_hbm.at[pl.ds(sid * _i32(HROWS), HROWS)])

    @pl.when(cid == 1)
    def _():
        pltpu.sync_copy(
            src,
            acc_hbm.at[pl.ds(_i32(HALF) + sid * _i32(HROWS), HROWS)])


@functools.partial(
    pl.kernel,
    out_type=jax.ShapeDtypeStruct((NP, C), jnp.float32),
    mesh=_mesh,
    compiler_params=_sc_params,
    scratch_types=[
        pltpu.VMEM((SB, 128), jnp.int32),
        pltpu.VMEM((ZB, C), jnp.float32),
        pltpu.VMEM((128, C), jnp.float32),
        pltpu.VMEM_SHARED((HALF + 8, C), jnp.float32),
    ],
)
def _deg_sc(dst0_hbm, dst1_hbm, acc_hbm, dstv, zbuf, ones, acc_sh):
    cid = lax.axis_index("c")
    sid = lax.axis_index("s")
    _zero_acc(acc_sh, zbuf, sid)

    @_loop32(0, 128)
    def _(i):
        ones[pl.ds(i, 1), :] = jnp.ones((1, C), jnp.float32)

    plsc.subcore_barrier()

    @_loop32(0, NCH)
    def _(ch):
        base = (sid * _i32(NCH) + ch) * _i32(SB)
        _load_dst(dst0_hbm, dst1_hbm, dstv, base, cid)

        @_loop32(0, SB)
        def _(k):
            pltpu.sync_copy(ones, acc_sh.at[dstv.at[k]], add=True)

    plsc.subcore_barrier()
    _writeback(acc_sh, acc_hbm, cid, sid)


@functools.partial(
    pl.kernel,
    out_type=jax.ShapeDtypeStruct((NP, C), jnp.float32),
    mesh=_mesh,
    compiler_params=_sc_params,
    scratch_types=[
        pltpu.VMEM((SB, 128), jnp.int32),
        pltpu.VMEM((SB, 128), jnp.int32),
        pltpu.VMEM((128, C), jnp.float32),
        pltpu.VMEM((ZB, C), jnp.float32),
        pltpu.VMEM_SHARED((HALF + 8, C), jnp.float32),
        pltpu.SemaphoreType.DMA,
    ],
)
def _prop_sc(z_hbm, srcs_hbm, dst0_hbm, dst1_hbm, acc_hbm, srcv, dstv, rows,
             zbuf, acc_sh, sem):
    cid = lax.axis_index("c")
    sid = lax.axis_index("s")
    _zero_acc(acc_sh, zbuf, sid)
    plsc.subcore_barrier()

    @_loop32(0, NCH)
    def _(ch):
        base = (sid * _i32(NCH) + ch) * _i32(SB)
        pltpu.sync_copy(srcs_hbm.at[pl.ds(base, SB)], srcv)
        _load_dst(dst0_hbm, dst1_hbm, dstv, base, cid)

        @_loop32(0, SB)
        def _(k):
            pltpu.async_copy(z_hbm.at[srcv.at[k]], rows, sem).wait()
            pltpu.sync_copy(rows, acc_sh.at[dstv.at[k]], add=True)

    plsc.subcore_barrier()
    _writeback(acc_sh, acc_hbm, cid, sid)


RM = 3128


def _mlp_body(x_ref, w1_ref, b1_ref, w2_ref, b2_ref, h_ref):
    i = pl.program_id(0)
    h1 = jnp.maximum(
        jnp.dot(x_ref[...], w1_ref[...],
                preferred_element_type=jnp.float32) + b1_ref[...], 0.0)
    h2 = jnp.maximum(
        jnp.dot(h1, w2_ref[...],
                preferred_element_type=jnp.float32) + b2_ref[...], 0.0)
    rows = i * RM + lax.broadcasted_iota(jnp.int32, (RM, 1), 0)
    h_ref[...] = jnp.where(rows < N, h2, 0.0)


def _mlp(x_p, W1, b1, W2, b2):
    return pl.pallas_call(
        _mlp_body,
        grid=(NP // RM,),
        in_specs=[
            pl.BlockSpec((RM, D), lambda i: (i, _Z)),
            pl.BlockSpec((D, H), lambda i: (_Z, _Z)),
            pl.BlockSpec((1, H), lambda i: (_Z, _Z)),
            pl.BlockSpec((H, C), lambda i: (_Z, _Z)),
            pl.BlockSpec((1, C), lambda i: (_Z, _Z)),
        ],
        out_specs=pl.BlockSpec((RM, C), lambda i: (i, _Z)),
        out_shape=jax.ShapeDtypeStruct((NP, C), jnp.float32),
    )(x_p, W1, b1.reshape(1, H), W2, b2.reshape(1, C))


RP = 6256


def _prep_body(dg_ref, h_ref, dis_ref, d2_ref, z_ref):
    i = pl.program_id(0)
    deg = dg_ref[:, 0:1] + 1.0
    rows = i * RP + lax.broadcasted_iota(jnp.int32, (RP, 1), 0)
    valid = rows < N
    dis = jnp.where(valid, lax.rsqrt(deg), 0.0)
    dis_ref[...] = dis
    d2_ref[...] = jnp.where(valid, 1.0 / deg, 0.0)
    z_ref[...] = dis * h_ref[...]


def _prep(deg_acc, h):
    return pl.pallas_call(
        _prep_body,
        grid=(NP // RP,),
        in_specs=[pl.BlockSpec((RP, C), lambda i: (i, _Z))] * 2,
        out_specs=[
            pl.BlockSpec((RP, 1), lambda i: (i, _Z)),
            pl.BlockSpec((RP, 1), lambda i: (i, _Z)),
            pl.BlockSpec((RP, C), lambda i: (i, _Z)),
        ],
        out_shape=[
            jax.ShapeDtypeStruct((NP, 1), jnp.float32),
            jax.ShapeDtypeStruct((NP, 1), jnp.float32),
            jax.ShapeDtypeStruct((NP, C), jnp.float32),
        ],
    )(deg_acc, h)


def _combine_body(ag_ref, o_ref, h_ref, dis_ref, d2_ref, on_ref, zn_ref):
    on = (1.0 - ALPHA) * (dis_ref[...] * ag_ref[...]
                          + d2_ref[...] * o_ref[...]) + ALPHA * h_ref[...]
    on_ref[...] = on
    zn_ref[...] = dis_ref[...] * on


def _combine(agg, out, h, dis, d2):
    return pl.pallas_call(
        _combine_body,
        grid=(NP // RP,),
        in_specs=[
            pl.BlockSpec((RP, C), lambda i: (i, _Z)),
            pl.BlockSpec((RP, C), lambda i: (i, _Z)),
            pl.BlockSpec((RP, C), lambda i: (i, _Z)),
            pl.BlockSpec((RP, 1), lambda i: (i, _Z)),
            pl.BlockSpec((RP, 1), lambda i: (i, _Z)),
        ],
        out_specs=[
            pl.BlockSpec((RP, C), lambda i: (i, _Z)),
            pl.BlockSpec((RP, C), lambda i: (i, _Z)),
        ],
        out_shape=[
            jax.ShapeDtypeStruct((NP, C), jnp.float32),
            jax.ShapeDtypeStruct((NP, C), jnp.float32),
        ],
    )(agg, out, h, dis, d2)


def kernel(x, edge_index, W1, b1, W2, b2):
    x = x.astype(jnp.float32)
    ei = edge_index.astype(jnp.int32)
    pad = jnp.full((EP - E,), N, jnp.int32)
    srcs = jnp.concatenate([ei[0], pad]).reshape(NS * TT, 128)
    dst = jnp.concatenate([ei[1], pad])
    dst0 = jnp.where(dst < HALF, dst, DUMP).astype(jnp.int32)
    dst1 = jnp.where((dst >= HALF) & (dst < N), dst - HALF,
                     DUMP).astype(jnp.int32)
    dst0 = dst0.reshape(NS * TT, 128)
    dst1 = dst1.reshape(NS * TT, 128)
    x_p = jnp.pad(x, ((0, NP - N), (0, 0)))

    h = _mlp(x_p, W1.astype(jnp.float32), b1.astype(jnp.float32),
             W2.astype(jnp.float32), b2.astype(jnp.float32))
    deg_acc = _deg_sc(dst0, dst1)
    dis, d2, z = _prep(deg_acc, h)

    out = h
    for _ in range(K):
        agg = _prop_sc(z, srcs, dst0, dst1)
        out, z = _combine(agg, out, h, dis, d2)
    return out[:N].astype(jnp.float64)


# speedup vs baseline: 357.2516x; 1.5629x over previous
import functools

import jax
import jax.numpy as jnp
import numpy as np
from jax import lax
from jax.experimental import pallas as pl
from jax.experimental.pallas import tpu as pltpu
from jax.experimental.pallas import tpu_sc as plsc

N = 100000
E = 3200000
D = 128
H = 64
C = 16
K = 10
ALPHA = 0.1

NP = 100096
HALF = NP // 2
HROWS = HALF // 16
ZB = 184

NS = 16
GW = 1024
TT = 200
EP = NS * TT * GW
SB = 8
NCH = TT // SB

_mesh = plsc.VectorSubcoreMesh(core_axis_name="c", subcore_axis_name="s")
_sc_params = pltpu.CompilerParams(use_tc_tiling_on_sc=False)
_Z = np.int32(0)


def _i32(v):
    return jnp.asarray(v, jnp.int32)


def _loop32(lo, hi):
    return pl.loop(jnp.int32(lo), jnp.int32(hi))


def _zero_acc(acc_sh, zbuf, sid):
    @_loop32(0, ZB)
    def _(i):
        zbuf[pl.ds(i, 1), :] = jnp.zeros((1, C), jnp.float32)

    @_loop32(0, 17)
    def _(j):
        pltpu.sync_copy(zbuf,
                        acc_sh.at[pl.ds(sid * _i32(HROWS) + j * _i32(ZB),
                                        ZB)])


def _load_dst(dst0_hbm, dst1_hbm, dstv, base, cid):
    @pl.when(cid == 0)
    def _():
        pltpu.sync_copy(dst0_hbm.at[pl.ds(base, SB)], dstv)

    @pl.when(cid == 1)
    def _():
        pltpu.sync_copy(dst1_hbm.at[pl.ds(base, SB)], dstv)


def _writeback(acc_sh, acc_hbm, cid, sid):
    src = acc_sh.at[pl.ds(sid * _i32(HROWS), HROWS)]

    @pl.when(cid == 0)
    def _():
        pltpu.sync_copy(src, acc_hbm.at[pl.ds(sid * _i32(HROWS), HROWS)])

    @pl.when(cid == 1)
    def _():
        pltpu.sync_copy(
            src,
            acc_hbm.at[pl.ds(_i32(HALF) + sid * _i32(HROWS), HROWS)])


@functools.partial(
    pl.kernel,
    out_type=jax.ShapeDtypeStruct((NP, C), jnp.float32),
    mesh=_mesh,
    compiler_params=_sc_params,
    scratch_types=[
        pltpu.VMEM((SB, GW), jnp.int32),
        pltpu.VMEM((ZB, C), jnp.float32),
        pltpu.VMEM((GW, C), jnp.float32),
        pltpu.VMEM_SHARED((HALF + NS, C), jnp.float32),
    ],
)
def _deg_sc(dst0_hbm, dst1_hbm, acc_hbm, dstv, zbuf, ones, acc_sh):
    cid = lax.axis_index("c")
    sid = lax.axis_index("s")
    _zero_acc(acc_sh, zbuf, sid)

    @_loop32(0, GW)
    def _(i):
        ones[pl.ds(i, 1), :] = jnp.ones((1, C), jnp.float32)

    plsc.subcore_barrier()

    @_loop32(0, NCH)
    def _(ch):
        base = (sid * _i32(NCH) + ch) * _i32(SB)
        _load_dst(dst0_hbm, dst1_hbm, dstv, base, cid)

        @_loop32(0, SB)
        def _(g):
            pltpu.sync_copy(ones, acc_sh.at[dstv.at[g]], add=True)

    plsc.subcore_barrier()
    _writeback(acc_sh, acc_hbm, cid, sid)


@functools.partial(
    pl.kernel,
    out_type=jax.ShapeDtypeStruct((NP, C), jnp.float32),
    mesh=_mesh,
    compiler_params=_sc_params,
    scratch_types=[
        pltpu.VMEM((SB, GW), jnp.int32),
        pltpu.VMEM((SB, GW), jnp.int32),
        pltpu.VMEM((GW, C), jnp.float32),
        pltpu.VMEM((ZB, C), jnp.float32),
        pltpu.VMEM_SHARED((HALF + NS, C), jnp.float32),
        pltpu.SemaphoreType.DMA,
    ],
)
def _prop_sc(z_hbm, srcs_hbm, dst0_hbm, dst1_hbm, acc_hbm, srcv, dstv, rows,
             zbuf, acc_sh, sem):
    cid = lax.axis_index("c")
    sid = lax.axis_index("s")
    _zero_acc(acc_sh, zbuf, sid)
    plsc.subcore_barrier()

    @_loop32(0, NCH)
    def _(ch):
        base = (sid * _i32(NCH) + ch) * _i32(SB)
        pltpu.sync_copy(srcs_hbm.at[pl.ds(base, SB)], srcv)
        _load_dst(dst0_hbm, dst1_hbm, dstv, base, cid)

        @_loop32(0, SB)
        def _(g):
            pltpu.async_copy(z_hbm.at[srcv.at[g]], rows, sem).wait()
            pltpu.sync_copy(rows, acc_sh.at[dstv.at[g]], add=True)

    plsc.subcore_barrier()
    _writeback(acc_sh, acc_hbm, cid, sid)


RM = 3128


def _mlp_body(x_ref, w1_ref, b1_ref, w2_ref, b2_ref, h_ref):
    i = pl.program_id(0)
    h1 = jnp.maximum(
        jnp.dot(x_ref[...], w1_ref[...],
                preferred_element_type=jnp.float32) + b1_ref[...], 0.0)
    h2 = jnp.maximum(
        jnp.dot(h1, w2_ref[...],
                preferred_element_type=jnp.float32) + b2_ref[...], 0.0)
    rows = i * RM + lax.broadcasted_iota(jnp.int32, (RM, 1), 0)
    h_ref[...] = jnp.where(rows < N, h2, 0.0)


def _mlp(x_p, W1, b1, W2, b2):
    return pl.pallas_call(
        _mlp_body,
        grid=(NP // RM,),
        in_specs=[
            pl.BlockSpec((RM, D), lambda i: (i, _Z)),
            pl.BlockSpec((D, H), lambda i: (_Z, _Z)),
            pl.BlockSpec((1, H), lambda i: (_Z, _Z)),
            pl.BlockSpec((H, C), lambda i: (_Z, _Z)),
            pl.BlockSpec((1, C), lambda i: (_Z, _Z)),
        ],
        out_specs=pl.BlockSpec((RM, C), lambda i: (i, _Z)),
        out_shape=jax.ShapeDtypeStruct((NP, C), jnp.float32),
    )(x_p, W1, b1.reshape(1, H), W2, b2.reshape(1, C))


RP = 6256


def _prep_body(dg_ref, h_ref, dis_ref, d2_ref, z_ref):
    i = pl.program_id(0)
    deg = dg_ref[:, 0:1] + 1.0
    rows = i * RP + lax.broadcasted_iota(jnp.int32, (RP, 1), 0)
    valid = rows < N
    dis = jnp.where(valid, lax.rsqrt(deg), 0.0)
    dis_ref[...] = dis
    d2_ref[...] = jnp.where(valid, 1.0 / deg, 0.0)
    z_ref[...] = dis * h_ref[...]


def _prep(deg_acc, h):
    return pl.pallas_call(
        _prep_body,
        grid=(NP // RP,),
        in_specs=[pl.BlockSpec((RP, C), lambda i: (i, _Z))] * 2,
        out_specs=[
            pl.BlockSpec((RP, 1), lambda i: (i, _Z)),
            pl.BlockSpec((RP, 1), lambda i: (i, _Z)),
            pl.BlockSpec((RP, C), lambda i: (i, _Z)),
        ],
        out_shape=[
            jax.ShapeDtypeStruct((NP, 1), jnp.float32),
            jax.ShapeDtypeStruct((NP, 1), jnp.float32),
            jax.ShapeDtypeStruct((NP, C), jnp.float32),
        ],
    )(deg_acc, h)


def _combine_body(ag_ref, o_ref, h_ref, dis_ref, d2_ref, on_ref, zn_ref):
    on = (1.0 - ALPHA) * (dis_ref[...] * ag_ref[...]
                          + d2_ref[...] * o_ref[...]) + ALPHA * h_ref[...]
    on_ref[...] = on
    zn_ref[...] = dis_ref[...] * on


def _combine(agg, out, h, dis, d2):
    return pl.pallas_call(
        _combine_body,
        grid=(NP // RP,),
        in_specs=[
            pl.BlockSpec((RP, C), lambda i: (i, _Z)),
            pl.BlockSpec((RP, C), lambda i: (i, _Z)),
            pl.BlockSpec((RP, C), lambda i: (i, _Z)),
            pl.BlockSpec((RP, 1), lambda i: (i, _Z)),
            pl.BlockSpec((RP, 1), lambda i: (i, _Z)),
        ],
        out_specs=[
            pl.BlockSpec((RP, C), lambda i: (i, _Z)),
            pl.BlockSpec((RP, C), lambda i: (i, _Z)),
        ],
        out_shape=[
            jax.ShapeDtypeStruct((NP, C), jnp.float32),
            jax.ShapeDtypeStruct((NP, C), jnp.float32),
        ],
    )(agg, out, h, dis, d2)


def kernel(x, edge_index, W1, b1, W2, b2):
    x = x.astype(jnp.float32)
    ei = edge_index.astype(jnp.int32)
    pad = jnp.full((EP - E,), N, jnp.int32)
    srcs = jnp.concatenate([ei[0], pad]).reshape(NS * TT, GW)
    dst = jnp.concatenate([ei[1], pad])
    dumpr = _i32(HALF) + jnp.arange(EP, dtype=jnp.int32) // (GW * TT)
    dst0 = jnp.where(dst < HALF, dst, dumpr).astype(jnp.int32)
    dst1 = jnp.where((dst >= HALF) & (dst < N), dst - HALF,
                     dumpr).astype(jnp.int32)
    dst0 = dst0.reshape(NS * TT, GW)
    dst1 = dst1.reshape(NS * TT, GW)
    x_p = jnp.pad(x, ((0, NP - N), (0, 0)))

    h = _mlp(x_p, W1.astype(jnp.float32), b1.astype(jnp.float32),
             W2.astype(jnp.float32), b2.astype(jnp.float32))
    deg_acc = _deg_sc(dst0, dst1)
    dis, d2, z = _prep(deg_acc, h)

    out = h
    for _ in range(K):
        agg = _prop_sc(z, srcs, dst0, dst1)
        out, z = _combine(agg, out, h, dis, d2)
    return out[:N].astype(jnp.float64)


# speedup vs baseline: 360.6714x; 1.0096x over previous
import functools

import jax
import jax.numpy as jnp
import numpy as np
from jax import lax
from jax.experimental import pallas as pl
from jax.experimental.pallas import tpu as pltpu
from jax.experimental.pallas import tpu_sc as plsc

N = 100000
E = 3200000
D = 128
H = 64
C = 16
K = 10
ALPHA = 0.1

NP = 100096
HALF = NP // 2
HROWS = HALF // 16
ZB = 184

NS = 16
GW = 512
TT = 400
EP = NS * TT * GW
SB = 8
NCH = TT // SB

_mesh = plsc.VectorSubcoreMesh(core_axis_name="c", subcore_axis_name="s")
_sc_params = pltpu.CompilerParams(use_tc_tiling_on_sc=False)
_Z = np.int32(0)


def _i32(v):
    return jnp.asarray(v, jnp.int32)


def _loop32(lo, hi):
    return pl.loop(jnp.int32(lo), jnp.int32(hi))


def _zero_acc(acc_sh, zbuf, sid):
    @_loop32(0, ZB)
    def _(i):
        zbuf[pl.ds(i, 1), :] = jnp.zeros((1, C), jnp.float32)

    @_loop32(0, 17)
    def _(j):
        pltpu.sync_copy(zbuf,
                        acc_sh.at[pl.ds(sid * _i32(HROWS) + j * _i32(ZB),
                                        ZB)])


def _load_dst(dst0_hbm, dst1_hbm, dstv, base, cid):
    @pl.when(cid == 0)
    def _():
        pltpu.sync_copy(dst0_hbm.at[pl.ds(base, SB)], dstv)

    @pl.when(cid == 1)
    def _():
        pltpu.sync_copy(dst1_hbm.at[pl.ds(base, SB)], dstv)


def _writeback(acc_sh, acc_hbm, cid, sid):
    src = acc_sh.at[pl.ds(sid * _i32(HROWS), HROWS)]

    @pl.when(cid == 0)
    def _():
        pltpu.sync_copy(src, acc_hbm.at[pl.ds(sid * _i32(HROWS), HROWS)])

    @pl.when(cid == 1)
    def _():
        pltpu.sync_copy(
            src,
            acc_hbm.at[pl.ds(_i32(HALF) + sid * _i32(HROWS), HROWS)])


@functools.partial(
    pl.kernel,
    out_type=jax.ShapeDtypeStruct((NP, C), jnp.float32),
    mesh=_mesh,
    compiler_params=_sc_params,
    scratch_types=[
        pltpu.VMEM((SB, GW), jnp.int32),
        pltpu.VMEM((ZB, C), jnp.float32),
        pltpu.VMEM((GW, C), jnp.float32),
        pltpu.VMEM_SHARED((HALF + NS, C), jnp.float32),
        pltpu.SemaphoreType.DMA,
    ],
)
def _deg_sc(dst0_hbm, dst1_hbm, acc_hbm, dstv, zbuf, ones, acc_sh, sem):
    cid = lax.axis_index("c")
    sid = lax.axis_index("s")
    _zero_acc(acc_sh, zbuf, sid)

    @_loop32(0, GW)
    def _(i):
        ones[pl.ds(i, 1), :] = jnp.ones((1, C), jnp.float32)

    plsc.subcore_barrier()

    @_loop32(0, NCH)
    def _(ch):
        base = (sid * _i32(NCH) + ch) * _i32(SB)
        _load_dst(dst0_hbm, dst1_hbm, dstv, base, cid)

        cps = [
            pltpu.async_copy(ones, acc_sh.at[dstv.at[_i32(b)]], sem,
                             add=True) for b in range(SB)
        ]
        for cp in cps:
            cp.wait()

    plsc.subcore_barrier()
    _writeback(acc_sh, acc_hbm, cid, sid)


@functools.partial(
    pl.kernel,
    out_type=jax.ShapeDtypeStruct((NP, C), jnp.float32),
    mesh=_mesh,
    compiler_params=_sc_params,
    scratch_types=[
        pltpu.VMEM((SB, GW), jnp.int32),
        pltpu.VMEM((SB, GW), jnp.int32),
        pltpu.VMEM((SB * GW, C), jnp.float32),
        pltpu.VMEM((ZB, C), jnp.float32),
        pltpu.VMEM_SHARED((HALF + NS, C), jnp.float32),
        pltpu.SemaphoreType.DMA,
        pltpu.SemaphoreType.DMA,
    ],
)
def _prop_sc(z_hbm, srcs_hbm, dst0_hbm, dst1_hbm, acc_hbm, srcv, dstv, rows,
             zbuf, acc_sh, gsem, ssem):
    cid = lax.axis_index("c")
    sid = lax.axis_index("s")
    _zero_acc(acc_sh, zbuf, sid)
    plsc.subcore_barrier()

    @_loop32(0, NCH)
    def _(ch):
        base = (sid * _i32(NCH) + ch) * _i32(SB)
        pltpu.sync_copy(srcs_hbm.at[pl.ds(base, SB)], srcv)
        _load_dst(dst0_hbm, dst1_hbm, dstv, base, cid)

        gcps = [
            pltpu.async_copy(z_hbm.at[srcv.at[_i32(b)]],
                             rows.at[pl.ds(_i32(b * GW), GW)], gsem)
            for b in range(SB)
        ]
        for cp in gcps:
            cp.wait()
        scps = [
            pltpu.async_copy(rows.at[pl.ds(_i32(b * GW), GW)],
                             acc_sh.at[dstv.at[_i32(b)]], ssem, add=True)
            for b in range(SB)
        ]
        for cp in scps:
            cp.wait()

    plsc.subcore_barrier()
    _writeback(acc_sh, acc_hbm, cid, sid)


RM = 3128


def _mlp_body(x_ref, w1_ref, b1_ref, w2_ref, b2_ref, h_ref):
    i = pl.program_id(0)
    h1 = jnp.maximum(
        jnp.dot(x_ref[...], w1_ref[...],
                preferred_element_type=jnp.float32) + b1_ref[...], 0.0)
    h2 = jnp.maximum(
        jnp.dot(h1, w2_ref[...],
                preferred_element_type=jnp.float32) + b2_ref[...], 0.0)
    rows = i * RM + lax.broadcasted_iota(jnp.int32, (RM, 1), 0)
    h_ref[...] = jnp.where(rows < N, h2, 0.0)


def _mlp(x_p, W1, b1, W2, b2):
    return pl.pallas_call(
        _mlp_body,
        grid=(NP // RM,),
        in_specs=[
            pl.BlockSpec((RM, D), lambda i: (i, _Z)),
            pl.BlockSpec((D, H), lambda i: (_Z, _Z)),
            pl.BlockSpec((1, H), lambda i: (_Z, _Z)),
            pl.BlockSpec((H, C), lambda i: (_Z, _Z)),
            pl.BlockSpec((1, C), lambda i: (_Z, _Z)),
        ],
        out_specs=pl.BlockSpec((RM, C), lambda i: (i, _Z)),
        out_shape=jax.ShapeDtypeStruct((NP, C), jnp.float32),
    )(x_p, W1, b1.reshape(1, H), W2, b2.reshape(1, C))


RP = 6256


def _prep_body(dg_ref, h_ref, dis_ref, d2_ref, z_ref):
    i = pl.program_id(0)
    deg = dg_ref[:, 0:1] + 1.0
    rows = i * RP + lax.broadcasted_iota(jnp.int32, (RP, 1), 0)
    valid = rows < N
    dis = jnp.where(valid, lax.rsqrt(deg), 0.0)
    dis_ref[...] = dis
    d2_ref[...] = jnp.where(valid, 1.0 / deg, 0.0)
    z_ref[...] = dis * h_ref[...]


def _prep(deg_acc, h):
    return pl.pallas_call(
        _prep_body,
        grid=(NP // RP,),
        in_specs=[pl.BlockSpec((RP, C), lambda i: (i, _Z))] * 2,
        out_specs=[
            pl.BlockSpec((RP, 1), lambda i: (i, _Z)),
            pl.BlockSpec((RP, 1), lambda i: (i, _Z)),
            pl.BlockSpec((RP, C), lambda i: (i, _Z)),
        ],
        out_shape=[
            jax.ShapeDtypeStruct((NP, 1), jnp.float32),
            jax.ShapeDtypeStruct((NP, 1), jnp.float32),
            jax.ShapeDtypeStruct((NP, C), jnp.float32),
        ],
    )(deg_acc, h)


def _combine_body(ag_ref, o_ref, h_ref, dis_ref, d2_ref, on_ref, zn_ref):
    on = (1.0 - ALPHA) * (dis_ref[...] * ag_ref[...]
                          + d2_ref[...] * o_ref[...]) + ALPHA * h_ref[...]
    on_ref[...] = on
    zn_ref[...] = dis_ref[...] * on


def _combine(agg, out, h, dis, d2):
    return pl.pallas_call(
        _combine_body,
        grid=(NP // RP,),
        in_specs=[
            pl.BlockSpec((RP, C), lambda i: (i, _Z)),
            pl.BlockSpec((RP, C), lambda i: (i, _Z)),
            pl.BlockSpec((RP, C), lambda i: (i, _Z)),
            pl.BlockSpec((RP, 1), lambda i: (i, _Z)),
            pl.BlockSpec((RP, 1), lambda i: (i, _Z)),
        ],
        out_specs=[
            pl.BlockSpec((RP, C), lambda i: (i, _Z)),
            pl.BlockSpec((RP, C), lambda i: (i, _Z)),
        ],
        out_shape=[
            jax.ShapeDtypeStruct((NP, C), jnp.float32),
            jax.ShapeDtypeStruct((NP, C), jnp.float32),
        ],
    )(agg, out, h, dis, d2)


def kernel(x, edge_index, W1, b1, W2, b2):
    x = x.astype(jnp.float32)
    ei = edge_index.astype(jnp.int32)
    pad = jnp.full((EP - E,), N, jnp.int32)
    srcs = jnp.concatenate([ei[0], pad]).reshape(NS * TT, GW)
    dst = jnp.concatenate([ei[1], pad])
    dumpr = _i32(HALF) + jnp.arange(EP, dtype=jnp.int32) // (GW * TT)
    dst0 = jnp.where(dst < HALF, dst, dumpr).astype(jnp.int32)
    dst1 = jnp.where((dst >= HALF) & (dst < N), dst - HALF,
                     dumpr).astype(jnp.int32)
    dst0 = dst0.reshape(NS * TT, GW)
    dst1 = dst1.reshape(NS * TT, GW)
    x_p = jnp.pad(x, ((0, NP - N), (0, 0)))

    h = _mlp(x_p, W1.astype(jnp.float32), b1.astype(jnp.float32),
             W2.astype(jnp.float32), b2.astype(jnp.float32))
    deg_acc = _deg_sc(dst0, dst1)
    dis, d2, z = _prep(deg_acc, h)

    out = h
    for _ in range(K):
        agg = _prop_sc(z, srcs, dst0, dst1)
        out, z = _combine(agg, out, h, dis, d2)
    return out[:N].astype(jnp.float64)


# speedup vs baseline: 678.9636x; 1.8825x over previous
import functools

import jax
import jax.numpy as jnp
import numpy as np
from jax import lax
from jax.experimental import pallas as pl
from jax.experimental.pallas import tpu as pltpu
from jax.experimental.pallas import tpu_sc as plsc

N = 100000
E = 3200000
D = 128
H = 64
C = 16
K = 10
ALPHA = 0.1

NP = 100096
WROWS = NP // 16
ZB = 184

NS = 16
GW = 128
SB = 8
TT = 784
EP = 2 * NS * TT * GW
NCH = TT // SB

_mesh = plsc.VectorSubcoreMesh(core_axis_name="c", subcore_axis_name="s")
_sc_params = pltpu.CompilerParams(use_tc_tiling_on_sc=False)
_Z = np.int32(0)


def _i32(v):
    return jnp.asarray(v, jnp.int32)


def _loop32(lo, hi):
    return pl.loop(jnp.int32(lo), jnp.int32(hi))


def _zero_acc(acc_sh, zbuf, sid):
    @_loop32(0, ZB)
    def _(i):
        zbuf[pl.ds(i, 1), :] = jnp.zeros((1, C), jnp.float32)

    @_loop32(0, WROWS // ZB)
    def _(j):
        pltpu.sync_copy(zbuf,
                        acc_sh.at[pl.ds(sid * _i32(WROWS) + j * _i32(ZB),
                                        ZB)])


def _writeback(acc_sh, acc_hbm, cid, sid):
    pltpu.sync_copy(
        acc_sh.at[pl.ds(sid * _i32(WROWS), WROWS)],
        acc_hbm.at[pl.ds(cid * _i32(NP) + sid * _i32(WROWS), WROWS)])


@functools.partial(
    pl.kernel,
    out_type=jax.ShapeDtypeStruct((2 * NP, C), jnp.float32),
    mesh=_mesh,
    compiler_params=_sc_params,
    scratch_types=[
        pltpu.VMEM((SB, GW), jnp.int32),
        pltpu.VMEM((ZB, C), jnp.float32),
        pltpu.VMEM((GW, C), jnp.float32),
        pltpu.VMEM_SHARED((NP, C), jnp.float32),
        pltpu.SemaphoreType.DMA,
    ],
)
def _deg_sc(dst_hbm, acc_hbm, dstv, zbuf, ones, acc_sh, sem):
    cid = lax.axis_index("c")
    sid = lax.axis_index("s")
    _zero_acc(acc_sh, zbuf, sid)

    @_loop32(0, GW)
    def _(i):
        ones[pl.ds(i, 1), :] = jnp.ones((1, C), jnp.float32)

    plsc.subcore_barrier()

    @_loop32(0, NCH)
    def _(ch):
        base = ((cid * _i32(NS) + sid) * _i32(NCH) + ch) * _i32(SB)
        pltpu.sync_copy(dst_hbm.at[pl.ds(base, SB)], dstv)

        cps = [
            pltpu.async_copy(ones, acc_sh.at[dstv.at[_i32(b)]], sem,
                             add=True) for b in range(SB)
        ]
        for cp in cps:
            cp.wait()

    plsc.subcore_barrier()
    _writeback(acc_sh, acc_hbm, cid, sid)


@functools.partial(
    pl.kernel,
    out_type=jax.ShapeDtypeStruct((2 * NP, C), jnp.float32),
    mesh=_mesh,
    compiler_params=_sc_params,
    scratch_types=[
        pltpu.VMEM((SB, GW), jnp.int32),
        pltpu.VMEM((SB, GW), jnp.int32),
        pltpu.VMEM((SB * GW, C), jnp.float32),
        pltpu.VMEM((ZB, C), jnp.float32),
        pltpu.VMEM_SHARED((NP, C), jnp.float32),
        pltpu.SemaphoreType.DMA,
        pltpu.SemaphoreType.DMA,
    ],
)
def _prop_sc(z_hbm, srcs_hbm, dst_hbm, acc_hbm, srcv, dstv, rows, zbuf,
             acc_sh, gsem, ssem):
    cid = lax.axis_index("c")
    sid = lax.axis_index("s")
    _zero_acc(acc_sh, zbuf, sid)
    plsc.subcore_barrier()

    @_loop32(0, NCH)
    def _(ch):
        base = ((cid * _i32(NS) + sid) * _i32(NCH) + ch) * _i32(SB)
        pltpu.sync_copy(srcs_hbm.at[pl.ds(base, SB)], srcv)
        pltpu.sync_copy(dst_hbm.at[pl.ds(base, SB)], dstv)

        gcps = [
            pltpu.async_copy(z_hbm.at[srcv.at[_i32(b)]],
                             rows.at[pl.ds(_i32(b * GW), GW)], gsem)
            for b in range(SB)
        ]
        for cp in gcps:
            cp.wait()
        scps = [
            pltpu.async_copy(rows.at[pl.ds(_i32(b * GW), GW)],
                             acc_sh.at[dstv.at[_i32(b)]], ssem, add=True)
            for b in range(SB)
        ]
        for cp in scps:
            cp.wait()

    plsc.subcore_barrier()
    _writeback(acc_sh, acc_hbm, cid, sid)


RM = 3128


def _mlp_body(x_ref, w1_ref, b1_ref, w2_ref, b2_ref, h_ref):
    i = pl.program_id(0)
    h1 = jnp.maximum(
        jnp.dot(x_ref[...], w1_ref[...],
                preferred_element_type=jnp.float32) + b1_ref[...], 0.0)
    h2 = jnp.maximum(
        jnp.dot(h1, w2_ref[...],
                preferred_element_type=jnp.float32) + b2_ref[...], 0.0)
    rows = i * RM + lax.broadcasted_iota(jnp.int32, (RM, 1), 0)
    h_ref[...] = jnp.where(rows < N, h2, 0.0)


def _mlp(x_p, W1, b1, W2, b2):
    return pl.pallas_call(
        _mlp_body,
        grid=(NP // RM,),
        in_specs=[
            pl.BlockSpec((RM, D), lambda i: (i, _Z)),
            pl.BlockSpec((D, H), lambda i: (_Z, _Z)),
            pl.BlockSpec((1, H), lambda i: (_Z, _Z)),
            pl.BlockSpec((H, C), lambda i: (_Z, _Z)),
            pl.BlockSpec((1, C), lambda i: (_Z, _Z)),
        ],
        out_specs=pl.BlockSpec((RM, C), lambda i: (i, _Z)),
        out_shape=jax.ShapeDtypeStruct((NP, C), jnp.float32),
    )(x_p, W1, b1.reshape(1, H), W2, b2.reshape(1, C))


RP = 6256
_NB = np.int32(NP // RP)


def _prep_body(dga_ref, dgb_ref, h_ref, dis_ref, d2_ref, z_ref):
    i = pl.program_id(0)
    deg = dga_ref[:, 0:1] + dgb_ref[:, 0:1] + 1.0
    rows = i * RP + lax.broadcasted_iota(jnp.int32, (RP, 1), 0)
    valid = rows < N
    dis = jnp.where(valid, lax.rsqrt(deg), 0.0)
    dis_ref[...] = dis
    d2_ref[...] = jnp.where(valid, 1.0 / deg, 0.0)
    z_ref[...] = dis * h_ref[...]


def _prep(deg_acc, h):
    return pl.pallas_call(
        _prep_body,
        grid=(NP // RP,),
        in_specs=[
            pl.BlockSpec((RP, C), lambda i: (i, _Z)),
            pl.BlockSpec((RP, C), lambda i: (i + _NB, _Z)),
            pl.BlockSpec((RP, C), lambda i: (i, _Z)),
        ],
        out_specs=[
            pl.BlockSpec((RP, 1), lambda i: (i, _Z)),
            pl.BlockSpec((RP, 1), lambda i: (i, _Z)),
            pl.BlockSpec((RP, C), lambda i: (i, _Z)),
        ],
        out_shape=[
            jax.ShapeDtypeStruct((NP, 1), jnp.float32),
            jax.ShapeDtypeStruct((NP, 1), jnp.float32),
            jax.ShapeDtypeStruct((NP, C), jnp.float32),
        ],
    )(deg_acc, deg_acc, h)


def _combine_body(aga_ref, agb_ref, o_ref, h_ref, dis_ref, d2_ref, on_ref,
                  zn_ref):
    agg = aga_ref[...] + agb_ref[...]
    on = (1.0 - ALPHA) * (dis_ref[...] * agg
                          + d2_ref[...] * o_ref[...]) + ALPHA * h_ref[...]
    on_ref[...] = on
    zn_ref[...] = dis_ref[...] * on


def _combine(agg, out, h, dis, d2):
    return pl.pallas_call(
        _combine_body,
        grid=(NP // RP,),
        in_specs=[
            pl.BlockSpec((RP, C), lambda i: (i, _Z)),
            pl.BlockSpec((RP, C), lambda i: (i + _NB, _Z)),
            pl.BlockSpec((RP, C), lambda i: (i, _Z)),
            pl.BlockSpec((RP, C), lambda i: (i, _Z)),
            pl.BlockSpec((RP, 1), lambda i: (i, _Z)),
            pl.BlockSpec((RP, 1), lambda i: (i, _Z)),
        ],
        out_specs=[
            pl.BlockSpec((RP, C), lambda i: (i, _Z)),
            pl.BlockSpec((RP, C), lambda i: (i, _Z)),
        ],
        out_shape=[
            jax.ShapeDtypeStruct((NP, C), jnp.float32),
            jax.ShapeDtypeStruct((NP, C), jnp.float32),
        ],
    )(agg, agg, out, h, dis, d2)


def kernel(x, edge_index, W1, b1, W2, b2):
    x = x.astype(jnp.float32)
    ei = edge_index.astype(jnp.int32)
    pad = jnp.full((EP - E,), N, jnp.int32)
    srcs = jnp.concatenate([ei[0], pad]).reshape(2 * NS * TT, GW)
    dst = jnp.concatenate([ei[1], pad]).reshape(2 * NS * TT, GW)
    x_p = jnp.pad(x, ((0, NP - N), (0, 0)))

    h = _mlp(x_p, W1.astype(jnp.float32), b1.astype(jnp.float32),
             W2.astype(jnp.float32), b2.astype(jnp.float32))
    deg_acc = _deg_sc(dst)
    dis, d2, z = _prep(deg_acc, h)

    out = h
    for _ in range(K):
        agg = _prop_sc(z, srcs, dst)
        out, z = _combine(agg, out, h, dis, d2)
    return out[:N].astype(jnp.float64)


# speedup vs baseline: 679.1149x; 1.0002x over previous
import functools

import jax
import jax.numpy as jnp
import numpy as np
from jax import lax
from jax.experimental import pallas as pl
from jax.experimental.pallas import tpu as pltpu
from jax.experimental.pallas import tpu_sc as plsc

N = 100000
E = 3200000
D = 128
H = 64
C = 16
K = 10
ALPHA = 0.1

NP = 100096
WROWS = NP // 16
ZB = 184

NS = 16
GW = 256
SB = 4
TT = 392
EP = 2 * NS * TT * GW
NCH = TT // SB

_mesh = plsc.VectorSubcoreMesh(core_axis_name="c", subcore_axis_name="s")
_sc_params = pltpu.CompilerParams(use_tc_tiling_on_sc=False)
_Z = np.int32(0)


def _i32(v):
    return jnp.asarray(v, jnp.int32)


def _loop32(lo, hi):
    return pl.loop(jnp.int32(lo), jnp.int32(hi))


def _zero_acc(acc_sh, zbuf, sid):
    @_loop32(0, ZB)
    def _(i):
        zbuf[pl.ds(i, 1), :] = jnp.zeros((1, C), jnp.float32)

    @_loop32(0, WROWS // ZB)
    def _(j):
        pltpu.sync_copy(zbuf,
                        acc_sh.at[pl.ds(sid * _i32(WROWS) + j * _i32(ZB),
                                        ZB)])


def _writeback(acc_sh, acc_hbm, cid, sid):
    pltpu.sync_copy(
        acc_sh.at[pl.ds(sid * _i32(WROWS), WROWS)],
        acc_hbm.at[pl.ds(cid * _i32(NP) + sid * _i32(WROWS), WROWS)])


@functools.partial(
    pl.kernel,
    out_type=jax.ShapeDtypeStruct((2 * NP, C), jnp.float32),
    mesh=_mesh,
    compiler_params=_sc_params,
    scratch_types=[
        pltpu.VMEM((SB, GW), jnp.int32),
        pltpu.VMEM((ZB, C), jnp.float32),
        pltpu.VMEM((GW, C), jnp.float32),
        pltpu.VMEM_SHARED((NP, C), jnp.float32),
        pltpu.SemaphoreType.DMA,
    ],
)
def _deg_sc(dst_hbm, acc_hbm, dstv, zbuf, ones, acc_sh, sem):
    cid = lax.axis_index("c")
    sid = lax.axis_index("s")
    _zero_acc(acc_sh, zbuf, sid)

    @_loop32(0, GW)
    def _(i):
        ones[pl.ds(i, 1), :] = jnp.ones((1, C), jnp.float32)

    plsc.subcore_barrier()

    @_loop32(0, NCH)
    def _(ch):
        base = ((cid * _i32(NS) + sid) * _i32(NCH) + ch) * _i32(SB)
        pltpu.sync_copy(dst_hbm.at[pl.ds(base, SB)], dstv)

        cps = [
            pltpu.async_copy(ones, acc_sh.at[dstv.at[_i32(b)]], sem,
                             add=True) for b in range(SB)
        ]
        for cp in cps:
            cp.wait()

    plsc.subcore_barrier()
    _writeback(acc_sh, acc_hbm, cid, sid)


@functools.partial(
    pl.kernel,
    out_type=jax.ShapeDtypeStruct((2 * NP, C), jnp.float32),
    mesh=_mesh,
    compiler_params=_sc_params,
    scratch_types=[
        pltpu.VMEM((SB, GW), jnp.int32),
        pltpu.VMEM((SB, GW), jnp.int32),
        pltpu.VMEM((SB * GW, C), jnp.float32),
        pltpu.VMEM((ZB, C), jnp.float32),
        pltpu.VMEM_SHARED((NP, C), jnp.float32),
        pltpu.SemaphoreType.DMA,
        pltpu.SemaphoreType.DMA,
    ],
)
def _prop_sc(z_hbm, srcs_hbm, dst_hbm, acc_hbm, srcv, dstv, rows, zbuf,
             acc_sh, gsem, ssem):
    cid = lax.axis_index("c")
    sid = lax.axis_index("s")
    _zero_acc(acc_sh, zbuf, sid)
    plsc.subcore_barrier()

    @_loop32(0, NCH)
    def _(ch):
        base = ((cid * _i32(NS) + sid) * _i32(NCH) + ch) * _i32(SB)
        pltpu.sync_copy(srcs_hbm.at[pl.ds(base, SB)], srcv)
        pltpu.sync_copy(dst_hbm.at[pl.ds(base, SB)], dstv)

        gcps = [
            pltpu.async_copy(z_hbm.at[srcv.at[_i32(b)]],
                             rows.at[pl.ds(_i32(b * GW), GW)], gsem)
            for b in range(SB)
        ]
        for cp in gcps:
            cp.wait()
        scps = [
            pltpu.async_copy(rows.at[pl.ds(_i32(b * GW), GW)],
                             acc_sh.at[dstv.at[_i32(b)]], ssem, add=True)
            for b in range(SB)
        ]
        for cp in scps:
            cp.wait()

    plsc.subcore_barrier()
    _writeback(acc_sh, acc_hbm, cid, sid)


RM = 3128


def _mlp_body(x_ref, w1_ref, b1_ref, w2_ref, b2_ref, h_ref):
    i = pl.program_id(0)
    h1 = jnp.maximum(
        jnp.dot(x_ref[...], w1_ref[...],
                preferred_element_type=jnp.float32) + b1_ref[...], 0.0)
    h2 = jnp.maximum(
        jnp.dot(h1, w2_ref[...],
                preferred_element_type=jnp.float32) + b2_ref[...], 0.0)
    rows = i * RM + lax.broadcasted_iota(jnp.int32, (RM, 1), 0)
    h_ref[...] = jnp.where(rows < N, h2, 0.0)


def _mlp(x_p, W1, b1, W2, b2):
    return pl.pallas_call(
        _mlp_body,
        grid=(NP // RM,),
        in_specs=[
            pl.BlockSpec((RM, D), lambda i: (i, _Z)),
            pl.BlockSpec((D, H), lambda i: (_Z, _Z)),
            pl.BlockSpec((1, H), lambda i: (_Z, _Z)),
            pl.BlockSpec((H, C), lambda i: (_Z, _Z)),
            pl.BlockSpec((1, C), lambda i: (_Z, _Z)),
        ],
        out_specs=pl.BlockSpec((RM, C), lambda i: (i, _Z)),
        out_shape=jax.ShapeDtypeStruct((NP, C), jnp.float32),
    )(x_p, W1, b1.reshape(1, H), W2, b2.reshape(1, C))


RP = 6256
_NB = np.int32(NP // RP)


def _prep_body(dga_ref, dgb_ref, h_ref, dis_ref, d2_ref, z_ref):
    i = pl.program_id(0)
    deg = dga_ref[:, 0:1] + dgb_ref[:, 0:1] + 1.0
    rows = i * RP + lax.broadcasted_iota(jnp.int32, (RP, 1), 0)
    valid = rows < N
    dis = jnp.where(valid, lax.rsqrt(deg), 0.0)
    dis_ref[...] = dis
    d2_ref[...] = jnp.where(valid, 1.0 / deg, 0.0)
    z_ref[...] = dis * h_ref[...]


def _prep(deg_acc, h):
    return pl.pallas_call(
        _prep_body,
        grid=(NP // RP,),
        in_specs=[
            pl.BlockSpec((RP, C), lambda i: (i, _Z)),
            pl.BlockSpec((RP, C), lambda i: (i + _NB, _Z)),
            pl.BlockSpec((RP, C), lambda i: (i, _Z)),
        ],
        out_specs=[
            pl.BlockSpec((RP, 1), lambda i: (i, _Z)),
            pl.BlockSpec((RP, 1), lambda i: (i, _Z)),
            pl.BlockSpec((RP, C), lambda i: (i, _Z)),
        ],
        out_shape=[
            jax.ShapeDtypeStruct((NP, 1), jnp.float32),
            jax.ShapeDtypeStruct((NP, 1), jnp.float32),
            jax.ShapeDtypeStruct((NP, C), jnp.float32),
        ],
    )(deg_acc, deg_acc, h)


def _combine_body(aga_ref, agb_ref, o_ref, h_ref, dis_ref, d2_ref, on_ref,
                  zn_ref):
    agg = aga_ref[...] + agb_ref[...]
    on = (1.0 - ALPHA) * (dis_ref[...] * agg
                          + d2_ref[...] * o_ref[...]) + ALPHA * h_ref[...]
    on_ref[...] = on
    zn_ref[...] = dis_ref[...] * on


def _combine(agg, out, h, dis, d2):
    return pl.pallas_call(
        _combine_body,
        grid=(NP // RP,),
        in_specs=[
            pl.BlockSpec((RP, C), lambda i: (i, _Z)),
            pl.BlockSpec((RP, C), lambda i: (i + _NB, _Z)),
            pl.BlockSpec((RP, C), lambda i: (i, _Z)),
            pl.BlockSpec((RP, C), lambda i: (i, _Z)),
            pl.BlockSpec((RP, 1), lambda i: (i, _Z)),
            pl.BlockSpec((RP, 1), lambda i: (i, _Z)),
        ],
        out_specs=[
            pl.BlockSpec((RP, C), lambda i: (i, _Z)),
            pl.BlockSpec((RP, C), lambda i: (i, _Z)),
        ],
        out_shape=[
            jax.ShapeDtypeStruct((NP, C), jnp.float32),
            jax.ShapeDtypeStruct((NP, C), jnp.float32),
        ],
    )(agg, agg, out, h, dis, d2)


def kernel(x, edge_index, W1, b1, W2, b2):
    x = x.astype(jnp.float32)
    ei = edge_index.astype(jnp.int32)
    pad = jnp.full((EP - E,), N, jnp.int32)
    srcs = jnp.concatenate([ei[0], pad]).reshape(2 * NS * TT, GW)
    dst = jnp.concatenate([ei[1], pad]).reshape(2 * NS * TT, GW)
    x_p = jnp.pad(x, ((0, NP - N), (0, 0)))

    h = _mlp(x_p, W1.astype(jnp.float32), b1.astype(jnp.float32),
             W2.astype(jnp.float32), b2.astype(jnp.float32))
    deg_acc = _deg_sc(dst)
    dis, d2, z = _prep(deg_acc, h)

    out = h
    for _ in range(K):
        agg = _prop_sc(z, srcs, dst)
        out, z = _combine(agg, out, h, dis, d2)
    return out[:N].astype(jnp.float64)


# speedup vs baseline: 680.3273x; 1.0018x over previous
import functools

import jax
import jax.numpy as jnp
import numpy as np
from jax import lax
from jax.experimental import pallas as pl
from jax.experimental.pallas import tpu as pltpu
from jax.experimental.pallas import tpu_sc as plsc

N = 100000
E = 3200000
D = 128
H = 64
C = 16
K = 10
ALPHA = 0.1

NP = 100096
WROWS = NP // 16
ZB = 184

NS = 16
GW = 256
SB = 4
TT = 392
EP = 2 * NS * TT * GW
NCH = TT // SB

_mesh = plsc.VectorSubcoreMesh(core_axis_name="c", subcore_axis_name="s")
_sc_params = pltpu.CompilerParams(use_tc_tiling_on_sc=False)
_Z = np.int32(0)


def _i32(v):
    return jnp.asarray(v, jnp.int32)


def _loop32(lo, hi):
    return pl.loop(jnp.int32(lo), jnp.int32(hi))


def _zero_acc(acc_sh, zbuf, sid, sem):
    @_loop32(0, ZB)
    def _(i):
        zbuf[pl.ds(i, 1), :] = jnp.zeros((1, C), jnp.float32)

    cps = [
        pltpu.async_copy(
            zbuf, acc_sh.at[pl.ds(sid * _i32(WROWS) + _i32(j * ZB), ZB)],
            sem) for j in range(WROWS // ZB)
    ]
    for cp in cps:
        cp.wait()


def _writeback(acc_sh, acc_hbm, cid, sid):
    pltpu.sync_copy(
        acc_sh.at[pl.ds(sid * _i32(WROWS), WROWS)],
        acc_hbm.at[pl.ds(cid * _i32(NP) + sid * _i32(WROWS), WROWS)])


@functools.partial(
    pl.kernel,
    out_type=jax.ShapeDtypeStruct((2 * NP, C), jnp.float32),
    mesh=_mesh,
    compiler_params=_sc_params,
    scratch_types=[
        pltpu.VMEM((SB, GW), jnp.int32),
        pltpu.VMEM((ZB, C), jnp.float32),
        pltpu.VMEM((GW, C), jnp.float32),
        pltpu.VMEM_SHARED((NP, C), jnp.float32),
        pltpu.SemaphoreType.DMA,
    ],
)
def _deg_sc(dst_hbm, acc_hbm, dstv, zbuf, ones, acc_sh, sem):
    cid = lax.axis_index("c")
    sid = lax.axis_index("s")
    _zero_acc(acc_sh, zbuf, sid, sem)

    @_loop32(0, GW)
    def _(i):
        ones[pl.ds(i, 1), :] = jnp.ones((1, C), jnp.float32)

    plsc.subcore_barrier()

    @_loop32(0, NCH)
    def _(ch):
        base = ((cid * _i32(NS) + sid) * _i32(NCH) + ch) * _i32(SB)
        pltpu.sync_copy(dst_hbm.at[pl.ds(base, SB)], dstv)

        cps = [
            pltpu.async_copy(ones, acc_sh.at[dstv.at[_i32(b)]], sem,
                             add=True) for b in range(SB)
        ]
        for cp in cps:
            cp.wait()

    plsc.subcore_barrier()
    _writeback(acc_sh, acc_hbm, cid, sid)


@functools.partial(
    pl.kernel,
    out_type=jax.ShapeDtypeStruct((2 * NP, C), jnp.float32),
    mesh=_mesh,
    compiler_params=_sc_params,
    scratch_types=[
        pltpu.VMEM((SB, GW), jnp.int32),
        pltpu.VMEM((SB, GW), jnp.int32),
        pltpu.VMEM((SB * GW, C), jnp.float32),
        pltpu.VMEM((ZB, C), jnp.float32),
        pltpu.VMEM_SHARED((NP, C), jnp.float32),
        pltpu.SemaphoreType.DMA,
        pltpu.SemaphoreType.DMA,
    ],
)
def _prop_sc(z_hbm, srcs_hbm, dst_hbm, acc_hbm, srcv, dstv, rows, zbuf,
             acc_sh, gsem, ssem):
    cid = lax.axis_index("c")
    sid = lax.axis_index("s")
    _zero_acc(acc_sh, zbuf, sid, gsem)
    plsc.subcore_barrier()

    @_loop32(0, NCH)
    def _(ch):
        base = ((cid * _i32(NS) + sid) * _i32(NCH) + ch) * _i32(SB)
        pltpu.sync_copy(srcs_hbm.at[pl.ds(base, SB)], srcv)
        pltpu.sync_copy(dst_hbm.at[pl.ds(base, SB)], dstv)

        gcps = [
            pltpu.async_copy(z_hbm.at[srcv.at[_i32(b)]],
                             rows.at[pl.ds(_i32(b * GW), GW)], gsem)
            for b in range(SB)
        ]
        for cp in gcps:
            cp.wait()
        scps = [
            pltpu.async_copy(rows.at[pl.ds(_i32(b * GW), GW)],
                             acc_sh.at[dstv.at[_i32(b)]], ssem, add=True)
            for b in range(SB)
        ]
        for cp in scps:
            cp.wait()

    plsc.subcore_barrier()
    _writeback(acc_sh, acc_hbm, cid, sid)


RM = 3128


def _mlp_body(x_ref, w1_ref, b1_ref, w2_ref, b2_ref, h_ref):
    i = pl.program_id(0)
    h1 = jnp.maximum(
        jnp.dot(x_ref[...], w1_ref[...],
                preferred_element_type=jnp.float32) + b1_ref[...], 0.0)
    h2 = jnp.maximum(
        jnp.dot(h1, w2_ref[...],
                preferred_element_type=jnp.float32) + b2_ref[...], 0.0)
    rows = i * RM + lax.broadcasted_iota(jnp.int32, (RM, 1), 0)
    h_ref[...] = jnp.where(rows < N, h2, 0.0)


def _mlp(x_p, W1, b1, W2, b2):
    return pl.pallas_call(
        _mlp_body,
        grid=(NP // RM,),
        in_specs=[
            pl.BlockSpec((RM, D), lambda i: (i, _Z)),
            pl.BlockSpec((D, H), lambda i: (_Z, _Z)),
            pl.BlockSpec((1, H), lambda i: (_Z, _Z)),
            pl.BlockSpec((H, C), lambda i: (_Z, _Z)),
            pl.BlockSpec((1, C), lambda i: (_Z, _Z)),
        ],
        out_specs=pl.BlockSpec((RM, C), lambda i: (i, _Z)),
        out_shape=jax.ShapeDtypeStruct((NP, C), jnp.float32),
    )(x_p, W1, b1.reshape(1, H), W2, b2.reshape(1, C))


RP = 6256
_NB = np.int32(NP // RP)


def _prep_body(dga_ref, dgb_ref, h_ref, dis_ref, d2_ref, z_ref):
    i = pl.program_id(0)
    deg = dga_ref[:, 0:1] + dgb_ref[:, 0:1] + 1.0
    rows = i * RP + lax.broadcasted_iota(jnp.int32, (RP, 1), 0)
    valid = rows < N
    dis = jnp.where(valid, lax.rsqrt(deg), 0.0)
    dis_ref[...] = dis
    d2_ref[...] = jnp.where(valid, 1.0 / deg, 0.0)
    z_ref[...] = dis * h_ref[...]


def _prep(deg_acc, h):
    return pl.pallas_call(
        _prep_body,
        grid=(NP // RP,),
        in_specs=[
            pl.BlockSpec((RP, C), lambda i: (i, _Z)),
            pl.BlockSpec((RP, C), lambda i: (i + _NB, _Z)),
            pl.BlockSpec((RP, C), lambda i: (i, _Z)),
        ],
        out_specs=[
            pl.BlockSpec((RP, 1), lambda i: (i, _Z)),
            pl.BlockSpec((RP, 1), lambda i: (i, _Z)),
            pl.BlockSpec((RP, C), lambda i: (i, _Z)),
        ],
        out_shape=[
            jax.ShapeDtypeStruct((NP, 1), jnp.float32),
            jax.ShapeDtypeStruct((NP, 1), jnp.float32),
            jax.ShapeDtypeStruct((NP, C), jnp.float32),
        ],
    )(deg_acc, deg_acc, h)


def _combine_body(aga_ref, agb_ref, o_ref, h_ref, dis_ref, d2_ref, on_ref,
                  zn_ref):
    agg = aga_ref[...] + agb_ref[...]
    on = (1.0 - ALPHA) * (dis_ref[...] * agg
                          + d2_ref[...] * o_ref[...]) + ALPHA * h_ref[...]
    on_ref[...] = on
    zn_ref[...] = dis_ref[...] * on


def _combine(agg, out, h, dis, d2):
    return pl.pallas_call(
        _combine_body,
        grid=(NP // RP,),
        in_specs=[
            pl.BlockSpec((RP, C), lambda i: (i, _Z)),
            pl.BlockSpec((RP, C), lambda i: (i + _NB, _Z)),
            pl.BlockSpec((RP, C), lambda i: (i, _Z)),
            pl.BlockSpec((RP, C), lambda i: (i, _Z)),
            pl.BlockSpec((RP, 1), lambda i: (i, _Z)),
            pl.BlockSpec((RP, 1), lambda i: (i, _Z)),
        ],
        out_specs=[
            pl.BlockSpec((RP, C), lambda i: (i, _Z)),
            pl.BlockSpec((RP, C), lambda i: (i, _Z)),
        ],
        out_shape=[
            jax.ShapeDtypeStruct((NP, C), jnp.float32),
            jax.ShapeDtypeStruct((NP, C), jnp.float32),
        ],
    )(agg, agg, out, h, dis, d2)


def kernel(x, edge_index, W1, b1, W2, b2):
    x = x.astype(jnp.float32)
    ei = edge_index.astype(jnp.int32)
    pad = jnp.full((EP - E,), N, jnp.int32)
    srcs = jnp.concatenate([ei[0], pad]).reshape(2 * NS * TT, GW)
    dst = jnp.concatenate([ei[1], pad]).reshape(2 * NS * TT, GW)
    x_p = jnp.pad(x, ((0, NP - N), (0, 0)))

    h = _mlp(x_p, W1.astype(jnp.float32), b1.astype(jnp.float32),
             W2.astype(jnp.float32), b2.astype(jnp.float32))
    deg_acc = _deg_sc(dst)
    dis, d2, z = _prep(deg_acc, h)

    out = h
    for _ in range(K):
        agg = _prop_sc(z, srcs, dst)
        out, z = _combine(agg, out, h, dis, d2)
    return out[:N].astype(jnp.float64)
